# baseline (speedup 1.0000x reference)
import functools

import jax
import jax.numpy as jnp
from jax import lax
from jax.experimental import pallas as pl
from jax.experimental.pallas import tpu as pltpu
import jax.experimental.pallas.tpu_sc as plsc

_G = 64
_NUM_CORES = 2
_NUM_SUBCORES = 16
_NUM_TILES = _NUM_CORES * _NUM_SUBCORES


def _pad_rows(n):
  per_sub = -(-n // _NUM_SUBCORES)
  per_sub = -(-per_sub // 8) * 8
  return per_sub, per_sub * _NUM_SUBCORES


@functools.partial(jax.jit, static_argnames=("n_pad", "cpt", "h", "chunk"))
def _sc_segment_sum(x, src2d, dst1d, zeros_nh, *, n_pad, cpt, h, chunk):
  rows_per_sub = n_pad // _NUM_SUBCORES
  assert cpt % 2 == 0 and cpt >= 4

  mesh = plsc.VectorSubcoreMesh(core_axis_name="c", subcore_axis_name="s")

  @functools.partial(
      pl.kernel,
      out_type=jax.ShapeDtypeStruct((_NUM_CORES, n_pad, h), jnp.float32),
      mesh=mesh,
      scratch_types=[
          pltpu.VMEM((cpt, chunk), jnp.int32),
          pltpu.VMEM((chunk,), jnp.int32),
          pltpu.VMEM((chunk,), jnp.int32),
          pltpu.VMEM((chunk, h), jnp.float32),
          pltpu.VMEM((chunk, h), jnp.float32),
          pltpu.VMEM_SHARED((n_pad, h), jnp.float32),
          pltpu.SemaphoreType.DMA,
          pltpu.SemaphoreType.DMA,
          pltpu.SemaphoreType.DMA,
          pltpu.SemaphoreType.DMA,
      ],
  )
  def seg_kernel(x_hbm, src_hbm, dst_hbm1d, zeros_hbm, out_hbm,
                 sidx, didx0, didx1, rows0, rows1, acc_sh, g0, g1, s0, s1):
    c = lax.axis_index("c")
    s = lax.axis_index("s")
    wid = s * _NUM_CORES + c

    r0 = s * rows_per_sub
    pltpu.sync_copy(zeros_hbm.at[pl.ds(r0, rows_per_sub)],
                    acc_sh.at[pl.ds(r0, rows_per_sub)])

    pltpu.sync_copy(src_hbm.at[pl.ds(wid * cpt, cpt)], sidx)
    base = wid * cpt * chunk
    plsc.subcore_barrier()

    pltpu.sync_copy(dst_hbm1d.at[pl.ds(base, chunk)], didx0)
    pltpu.sync_copy(dst_hbm1d.at[pl.ds(base + chunk, chunk)], didx1)
    pltpu.async_copy(x_hbm.at[sidx.at[0]], rows0, g0)
    pltpu.async_copy(x_hbm.at[sidx.at[1]], rows1, g1)

    def pair(i, _):
      j = i * 2
      pltpu.make_async_copy(x_hbm.at[sidx.at[j]], rows0, g0).wait()
      pltpu.async_copy(rows0, acc_sh.at[didx0], s0, add=True)
      pltpu.make_async_copy(x_hbm.at[sidx.at[j + 1]], rows1, g1).wait()
      pltpu.async_copy(rows1, acc_sh.at[didx1], s1, add=True)
      pltpu.make_async_copy(rows0, acc_sh.at[didx0], s0).wait()
      pltpu.sync_copy(dst_hbm1d.at[pl.ds(base + (j + 2) * chunk, chunk)],
                      didx0)
      pltpu.async_copy(x_hbm.at[sidx.at[j + 2]], rows0, g0)
      pltpu.make_async_copy(rows1, acc_sh.at[didx1], s1).wait()
      pltpu.sync_copy(dst_hbm1d.at[pl.ds(base + (j + 3) * chunk, chunk)],
                      didx1)
      pltpu.async_copy(x_hbm.at[sidx.at[j + 3]], rows1, g1)
      return 0

    lax.fori_loop(0, cpt // 2 - 1, pair, 0)
    pltpu.make_async_copy(x_hbm.at[sidx.at[cpt - 2]], rows0, g0).wait()
    pltpu.sync_copy(rows0, acc_sh.at[didx0], add=True)
    pltpu.make_async_copy(x_hbm.at[sidx.at[cpt - 1]], rows1, g1).wait()
    pltpu.sync_copy(rows1, acc_sh.at[didx1], add=True)
    plsc.subcore_barrier()

    pltpu.sync_copy(acc_sh.at[pl.ds(r0, rows_per_sub)],
                    out_hbm.at[c, pl.ds(r0, rows_per_sub)])

  return seg_kernel(x, src2d, dst1d, zeros_nh)


@functools.partial(jax.jit, static_argnames=("n_pad", "e", "chunk"))
def _sc_degree_count(dst, ones_c, zeros_n, *, n_pad, e, chunk):
  per_tile = e // _NUM_TILES
  n_chunks = per_tile // chunk
  rows_per_sub = n_pad // _NUM_SUBCORES

  mesh = plsc.VectorSubcoreMesh(core_axis_name="c", subcore_axis_name="s")

  @functools.partial(
      pl.kernel,
      out_type=jax.ShapeDtypeStruct((_NUM_CORES * n_pad,), jnp.float32),
      mesh=mesh,
      scratch_types=[
          pltpu.VMEM((chunk,), jnp.int32),
          pltpu.VMEM((chunk,), jnp.float32),
          pltpu.VMEM((rows_per_sub,), jnp.float32),
          pltpu.VMEM_SHARED((n_pad,), jnp.float32),
      ],
  )
  def cnt_kernel(dst_hbm, ones_hbm, zeros_hbm, out_hbm, dst_v, ones_v,
                 bounce_v, acc_sh):
    c = lax.axis_index("c")
    s = lax.axis_index("s")
    wid = s * _NUM_CORES + c

    r0 = s * rows_per_sub
    pltpu.sync_copy(zeros_hbm.at[pl.ds(r0, rows_per_sub)], bounce_v)
    pltpu.sync_copy(bounce_v, acc_sh.at[pl.ds(r0, rows_per_sub)])
    pltpu.sync_copy(ones_hbm, ones_v)
    plsc.subcore_barrier()

    base = wid * per_tile

    def body(i, _):
      off = base + i * chunk
      pltpu.sync_copy(dst_hbm.at[pl.ds(off, chunk)], dst_v)
      pltpu.sync_copy(ones_v, acc_sh.at[dst_v], add=True)
      return 0

    lax.fori_loop(0, n_chunks, body, 0)
    plsc.subcore_barrier()

    pltpu.sync_copy(acc_sh.at[pl.ds(r0, rows_per_sub)], bounce_v)
    pltpu.sync_copy(bounce_v,
                    out_hbm.at[pl.ds(c * n_pad + r0, rows_per_sub)])

  return cnt_kernel(dst, ones_c, zeros_n)


def _tc_layer_body(h_ref, p_ref, inv_ref, w1_ref, b1_ref, g_ref, be_ref,
                   w2_ref, b2_ref, pool_ref, hout_ref, poolout_ref):
  n = h_ref.shape[0]
  agg = (p_ref[0, :n] + p_ref[1, :n]) * inv_ref[...]
  z = h_ref[...] + agg
  z = jnp.dot(z, w1_ref[...], preferred_element_type=jnp.float32) + b1_ref[...]
  mu = jnp.mean(z, axis=0, keepdims=True)
  var = jnp.mean((z - mu) * (z - mu), axis=0, keepdims=True)
  z = (z - mu) * lax.rsqrt(var + 1e-5) * g_ref[...] + be_ref[...]
  z = jnp.maximum(z, 0.0)
  z = jnp.dot(z, w2_ref[...], preferred_element_type=jnp.float32) + b2_ref[...]
  hout_ref[...] = z
  poolout_ref[...] = pool_ref[...] + z


def _tc_layer(h, parts, inv_cnt, p, pool_in, *, n, hdim):
  return pl.pallas_call(
      _tc_layer_body,
      out_shape=(
          jax.ShapeDtypeStruct((n, hdim), jnp.float32),
          jax.ShapeDtypeStruct((n, hdim), jnp.float32),
      ),
  )(h, parts, inv_cnt,
    p["W1"], p["b1"].reshape(1, -1), p["gamma"].reshape(1, -1),
    p["beta"].reshape(1, -1), p["W2"], p["b2"].reshape(1, -1), pool_in)


def _tc_invcnt_body(cnt_ref, out_ref):
  n = out_ref.shape[0]
  cnt = cnt_ref[0, :n] + cnt_ref[1, :n]
  out_ref[...] = 1.0 / jnp.maximum(cnt[:, None], 1.0)


def _tc_pool_body(np_ref, batch_ref, out_ref):
  b = batch_ref[...]
  gids = lax.broadcasted_iota(jnp.int32, (1, _G), 1)
  mask = (b == gids).astype(jnp.float32)
  s = lax.dot_general(mask, np_ref[...], (((0,), (0,)), ((), ())),
                      preferred_element_type=jnp.float32)
  cnt = jnp.sum(mask, axis=0)[:, None]
  out_ref[...] = s / jnp.maximum(cnt, 1.0)


def kernel(x, edge_index, batch, params):
  n, d = x.shape
  e = edge_index.shape[1]
  hdim = params[0]["W1"].shape[1]
  chunk = 128
  cnt_chunk = 80

  _, n_pad = _pad_rows(n)
  src = edge_index[0].astype(jnp.int32)
  dst = edge_index[1].astype(jnp.int32)
  zeros_nh = jnp.zeros((n_pad, hdim), jnp.float32)
  zeros_n = jnp.zeros((n_pad,), jnp.float32)
  ones_c = jnp.ones((cnt_chunk,), jnp.float32)

  cpt = -(-e // (_NUM_TILES * chunk))
  cpt += cpt % 2
  e_pad = _NUM_TILES * cpt * chunk
  src2d = jnp.concatenate(
      [src, jnp.zeros((e_pad - e,), jnp.int32)]).reshape(-1, chunk)
  dst1d = jnp.concatenate(
      [dst, jnp.full((e_pad - e,), n_pad - 1, jnp.int32)])

  cnt_parts = _sc_degree_count(dst, ones_c, zeros_n, n_pad=n_pad, e=e,
                               chunk=cnt_chunk).reshape(_NUM_CORES, n_pad)
  inv_cnt = pl.pallas_call(
      _tc_invcnt_body,
      out_shape=jax.ShapeDtypeStruct((n, 1), jnp.float32),
  )(cnt_parts)

  h = x
  pool = jnp.zeros((n, hdim), jnp.float32)
  for p in params:
    parts = _sc_segment_sum(h, src2d, dst1d, zeros_nh, n_pad=n_pad, cpt=cpt,
                            h=hdim, chunk=chunk)
    h, pool = _tc_layer(h, parts, inv_cnt, p, pool, n=n, hdim=hdim)

  gpool = pl.pallas_call(
      _tc_pool_body,
      out_shape=jax.ShapeDtypeStruct((_G, hdim), jnp.float32),
  )(pool, batch.astype(jnp.int32).reshape(n, 1))

  return (pool, gpool)


# speedup vs baseline: 2.5894x; 2.5894x over previous
import functools

import jax
import jax.numpy as jnp
from jax import lax
from jax.experimental import pallas as pl
from jax.experimental.pallas import tpu as pltpu
import jax.experimental.pallas.tpu_sc as plsc

_G = 64
_NUM_CORES = 2
_NUM_SUBCORES = 16
_NUM_TILES = _NUM_CORES * _NUM_SUBCORES


def _pad_rows(n):
  per_sub = -(-n // _NUM_SUBCORES)
  per_sub = -(-per_sub // 8) * 8
  return per_sub, per_sub * _NUM_SUBCORES


@functools.partial(jax.jit, static_argnames=("n_pad", "e", "h", "chunk"))
def _sc_segment_sum(x, src, dst, zeros_nh, *, n_pad, e, h, chunk):
  per_tile = e // _NUM_TILES
  n_chunks = per_tile // chunk
  rows_per_sub = n_pad // _NUM_SUBCORES

  mesh = plsc.VectorSubcoreMesh(core_axis_name="c", subcore_axis_name="s")

  assert n_chunks % 2 == 1, "pipelined loop expects an odd chunk count"
  n_pairs = (n_chunks - 1) // 2

  @functools.partial(
      pl.kernel,
      out_type=jax.ShapeDtypeStruct((_NUM_CORES, n_pad, h), jnp.float32),
      mesh=mesh,
      scratch_types=[
          pltpu.VMEM((chunk,), jnp.int32),
          pltpu.VMEM((chunk,), jnp.int32),
          pltpu.VMEM((chunk,), jnp.int32),
          pltpu.VMEM((chunk,), jnp.int32),
          pltpu.VMEM((chunk, h), jnp.float32),
          pltpu.VMEM((chunk, h), jnp.float32),
          pltpu.VMEM_SHARED((n_pad, h), jnp.float32),
          pltpu.SemaphoreType.DMA,
          pltpu.SemaphoreType.DMA,
      ],
  )
  def seg_kernel(x_hbm, src_hbm, dst_hbm, zeros_hbm, out_hbm,
                 src0, dst0, src1, dst1, rows0, rows1, acc_sh, sem0, sem1):
    c = lax.axis_index("c")
    s = lax.axis_index("s")
    wid = s * _NUM_CORES + c

    r0 = s * rows_per_sub
    pltpu.sync_copy(zeros_hbm.at[pl.ds(r0, rows_per_sub)],
                    acc_sh.at[pl.ds(r0, rows_per_sub)])
    plsc.subcore_barrier()

    base = wid * per_tile

    pltpu.sync_copy(src_hbm.at[pl.ds(base, chunk)], src0)
    pltpu.sync_copy(dst_hbm.at[pl.ds(base, chunk)], dst0)
    pltpu.async_copy(x_hbm.at[src0], rows0, sem0)

    def pair(i, _):
      off1 = base + (i * 2 + 1) * chunk
      pltpu.sync_copy(src_hbm.at[pl.ds(off1, chunk)], src1)
      pltpu.sync_copy(dst_hbm.at[pl.ds(off1, chunk)], dst1)
      pltpu.async_copy(x_hbm.at[src1], rows1, sem1)
      pltpu.make_async_copy(x_hbm.at[src0], rows0, sem0).wait()
      pltpu.sync_copy(rows0, acc_sh.at[dst0], add=True)

      off2 = base + (i * 2 + 2) * chunk
      pltpu.sync_copy(src_hbm.at[pl.ds(off2, chunk)], src0)
      pltpu.sync_copy(dst_hbm.at[pl.ds(off2, chunk)], dst0)
      pltpu.async_copy(x_hbm.at[src0], rows0, sem0)
      pltpu.make_async_copy(x_hbm.at[src1], rows1, sem1).wait()
      pltpu.sync_copy(rows1, acc_sh.at[dst1], add=True)
      return 0

    lax.fori_loop(0, n_pairs, pair, 0)
    pltpu.make_async_copy(x_hbm.at[src0], rows0, sem0).wait()
    pltpu.sync_copy(rows0, acc_sh.at[dst0], add=True)
    plsc.subcore_barrier()

    pltpu.sync_copy(acc_sh.at[pl.ds(r0, rows_per_sub)],
                    out_hbm.at[c, pl.ds(r0, rows_per_sub)])

  return seg_kernel(x, src, dst, zeros_nh)


@functools.partial(jax.jit, static_argnames=("n_pad", "e", "chunk"))
def _sc_degree_count(dst, ones_c, zeros_n, *, n_pad, e, chunk):
  per_tile = e // _NUM_TILES
  n_chunks = per_tile // chunk
  rows_per_sub = n_pad // _NUM_SUBCORES

  mesh = plsc.VectorSubcoreMesh(core_axis_name="c", subcore_axis_name="s")

  @functools.partial(
      pl.kernel,
      out_type=jax.ShapeDtypeStruct((_NUM_CORES * n_pad,), jnp.float32),
      mesh=mesh,
      scratch_types=[
          pltpu.VMEM((chunk,), jnp.int32),
          pltpu.VMEM((chunk,), jnp.float32),
          pltpu.VMEM((rows_per_sub,), jnp.float32),
          pltpu.VMEM_SHARED((n_pad,), jnp.float32),
      ],
  )
  def cnt_kernel(dst_hbm, ones_hbm, zeros_hbm, out_hbm, dst_v, ones_v,
                 bounce_v, acc_sh):
    c = lax.axis_index("c")
    s = lax.axis_index("s")
    wid = s * _NUM_CORES + c

    r0 = s * rows_per_sub
    pltpu.sync_copy(zeros_hbm.at[pl.ds(r0, rows_per_sub)], bounce_v)
    pltpu.sync_copy(bounce_v, acc_sh.at[pl.ds(r0, rows_per_sub)])
    pltpu.sync_copy(ones_hbm, ones_v)
    plsc.subcore_barrier()

    base = wid * per_tile

    def body(i, _):
      off = base + i * chunk
      pltpu.sync_copy(dst_hbm.at[pl.ds(off, chunk)], dst_v)
      pltpu.sync_copy(ones_v, acc_sh.at[dst_v], add=True)
      return 0

    lax.fori_loop(0, n_chunks, body, 0)
    plsc.subcore_barrier()

    pltpu.sync_copy(acc_sh.at[pl.ds(r0, rows_per_sub)], bounce_v)
    pltpu.sync_copy(bounce_v,
                    out_hbm.at[pl.ds(c * n_pad + r0, rows_per_sub)])

  return cnt_kernel(dst, ones_c, zeros_n)


def _tc_dense(h_ref, p_ref, cnt_ref, w1_ref, b1_ref, g_ref, be_ref,
              w2_ref, b2_ref):
  n = h_ref.shape[0]
  cnt = jnp.maximum(cnt_ref[0, :n] + cnt_ref[1, :n], 1.0)[:, None]
  agg = (p_ref[0, :n] + p_ref[1, :n]) / cnt
  z = h_ref[...] + agg
  z = jnp.dot(z, w1_ref[...], preferred_element_type=jnp.float32) + b1_ref[...]
  mu = jnp.mean(z, axis=0, keepdims=True)
  var = jnp.mean((z - mu) * (z - mu), axis=0, keepdims=True)
  z = (z - mu) * lax.rsqrt(var + 1e-5) * g_ref[...] + be_ref[...]
  z = jnp.maximum(z, 0.0)
  z = jnp.dot(z, w2_ref[...], preferred_element_type=jnp.float32) + b2_ref[...]
  return z


def _tc_layer0_body(h_ref, p_ref, cnt_ref, w1_ref, b1_ref, g_ref, be_ref,
                    w2_ref, b2_ref, hout_ref, poolout_ref):
  z = _tc_dense(h_ref, p_ref, cnt_ref, w1_ref, b1_ref, g_ref, be_ref,
                w2_ref, b2_ref)
  hout_ref[...] = z
  poolout_ref[...] = z


def _tc_layer_body(h_ref, p_ref, cnt_ref, w1_ref, b1_ref, g_ref, be_ref,
                   w2_ref, b2_ref, pool_ref, hout_ref, poolout_ref):
  z = _tc_dense(h_ref, p_ref, cnt_ref, w1_ref, b1_ref, g_ref, be_ref,
                w2_ref, b2_ref)
  hout_ref[...] = z
  poolout_ref[...] = pool_ref[...] + z


def _tc_last_body(h_ref, p_ref, cnt_ref, w1_ref, b1_ref, g_ref, be_ref,
                  w2_ref, b2_ref, pool_ref, batch_ref,
                  hout_ref, poolout_ref, gpool_ref):
  z = _tc_dense(h_ref, p_ref, cnt_ref, w1_ref, b1_ref, g_ref, be_ref,
                w2_ref, b2_ref)
  hout_ref[...] = z
  pool = pool_ref[...] + z
  poolout_ref[...] = pool
  b = batch_ref[...]
  gids = lax.broadcasted_iota(jnp.int32, (1, _G), 1)
  mask = (b == gids).astype(jnp.float32)
  sums = lax.dot_general(mask, pool, (((0,), (0,)), ((), ())),
                         preferred_element_type=jnp.float32)
  gcnt = jnp.sum(mask, axis=0)[:, None]
  gpool_ref[...] = sums / jnp.maximum(gcnt, 1.0)


def _tc_layer(h, parts, cnt_parts, p, pool_in, batch2d, *, n, hdim, kind):
  weights = (p["W1"], p["b1"].reshape(1, -1), p["gamma"].reshape(1, -1),
             p["beta"].reshape(1, -1), p["W2"], p["b2"].reshape(1, -1))
  nh = jax.ShapeDtypeStruct((n, hdim), jnp.float32)
  if kind == "first":
    return pl.pallas_call(
        _tc_layer0_body, out_shape=(nh, nh),
    )(h, parts, cnt_parts, *weights)
  if kind == "mid":
    return pl.pallas_call(
        _tc_layer_body, out_shape=(nh, nh),
    )(h, parts, cnt_parts, *weights, pool_in)
  return pl.pallas_call(
      _tc_last_body,
      out_shape=(nh, nh, jax.ShapeDtypeStruct((_G, hdim), jnp.float32)),
  )(h, parts, cnt_parts, *weights, pool_in, batch2d)


def kernel(x, edge_index, batch, params):
  n, d = x.shape
  e = edge_index.shape[1]
  hdim = params[0]["W1"].shape[1]
  chunk = 80

  _, n_pad = _pad_rows(n)
  src = edge_index[0].astype(jnp.int32)
  dst = edge_index[1].astype(jnp.int32)
  zeros_nh = jnp.zeros((n_pad, hdim), jnp.float32)
  zeros_n = jnp.zeros((n_pad,), jnp.float32)
  ones_c = jnp.ones((chunk,), jnp.float32)

  cnt_parts = _sc_degree_count(dst, ones_c, zeros_n, n_pad=n_pad, e=e,
                               chunk=chunk).reshape(_NUM_CORES, n_pad)
  batch2d = batch.astype(jnp.int32).reshape(n, 1)

  h = x
  pool = None
  n_layers = len(params)
  for i, p in enumerate(params):
    parts = _sc_segment_sum(h, src, dst, zeros_nh, n_pad=n_pad, e=e,
                            h=hdim, chunk=chunk)
    kind = "first" if i == 0 else ("last" if i == n_layers - 1 else "mid")
    out = _tc_layer(h, parts, cnt_parts, p, pool, batch2d,
                    n=n, hdim=hdim, kind=kind)
    if kind == "last":
      h, pool, gpool = out
    else:
      h, pool = out

  return (pool, gpool)
